# Initial kernel scaffold; baseline (speedup 1.0000x reference)
import jax
import jax.numpy as jnp
from jax.experimental import pallas as pl


def kernel(x, edge_index, batch, W1a, b1a, W1b, b1b, W2a, b2a, W2b, b2b):
    raise NotImplementedError("write your pallas kernel here")


# speedup vs baseline: 16.9340x; 16.9340x over previous
import functools

import jax
import jax.numpy as jnp
from jax import lax
from jax.experimental import pallas as pl
from jax.experimental.pallas import tpu as pltpu
from jax.experimental.pallas import tpu_sc as plsc

NC = 2
NS = 16
NW = NC * NS
LANES = 16

N = 10000
E = 320000
D = 128
H = 64
G = 256

EPW = 10240
EP = NW * EPW
NBLK = EPW // 128
NPAD = 10240
RB = 1000


def _mm_body(x_ref, w_ref, o_ref):
    o_ref[...] = jnp.dot(x_ref[...], w_ref[...],
                         preferred_element_type=jnp.float32)


def _tc_a(x, w1a):
    return pl.pallas_call(
        _mm_body,
        grid=(N // RB,),
        in_specs=[
            pl.BlockSpec((RB, D), lambda i: (i, 0)),
            pl.BlockSpec((D, H), lambda i: (0, 0)),
        ],
        out_specs=pl.BlockSpec((RB, H), lambda i: (i, 0)),
        out_shape=jax.ShapeDtypeStruct((N, H), jnp.float32),
    )(x, w1a)


def _sc_rows_body(src_hbm, dst_hbm, y_hbm, z_hbm, out_hbm,
                  srcf, dst2d, rows0, rows1, acc_sh, sem0, sem1):
    c = lax.axis_index("c")
    s = lax.axis_index("s")
    wid = c * NS + s
    pltpu.sync_copy(z_hbm, acc_sh.at[pl.ds(s * 640, 640)])
    pltpu.sync_copy(src_hbm.at[pl.ds(wid * EPW, EPW)], srcf)
    pltpu.sync_copy(dst_hbm.at[pl.ds(wid * NBLK, NBLK)], dst2d)
    plsc.subcore_barrier()
    rows = (rows0, rows1)
    sems = (sem0, sem1)
    cps = [None, None]
    cps[0] = pltpu.async_copy(y_hbm.at[srcf.at[pl.ds(0, 128)]],
                              rows0, sem0)
    for j in range(NBLK):
        if j + 1 < NBLK:
            b = (j + 1) % 2
            cps[b] = pltpu.async_copy(
                y_hbm.at[srcf.at[pl.ds((j + 1) * 128, 128)]],
                rows[b], sems[b])
        cps[j % 2].wait()
        pltpu.sync_copy(rows[j % 2], acc_sh.at[dst2d.at[j]], add=True)
    plsc.subcore_barrier()
    pltpu.sync_copy(acc_sh.at[pl.ds(s * 640, 640)],
                    out_hbm.at[c, pl.ds(s * 640, 640)])


def _sc_b(src_flat, dst2d, y, zb):
    mesh = plsc.VectorSubcoreMesh(core_axis_name="c", subcore_axis_name="s",
                                  num_cores=NC, num_subcores=NS)
    f = functools.partial(
        pl.kernel, mesh=mesh,
        compiler_params=pltpu.CompilerParams(use_tc_tiling_on_sc=False),
        out_type=jax.ShapeDtypeStruct((NC, NPAD, H), jnp.float32),
        scratch_types=[
            pltpu.VMEM((EPW,), jnp.int32),
            pltpu.VMEM((NBLK, 128), jnp.int32),
            pltpu.VMEM((128, H), jnp.float32),
            pltpu.VMEM((128, H), jnp.float32),
            pltpu.VMEM_SHARED((NPAD, H), jnp.float32),
            pltpu.SemaphoreType.DMA,
            pltpu.SemaphoreType.DMA,
        ],
    )(_sc_rows_body)
    return f(src_flat, dst2d, y, zb)


def _mlp_body(y_ref, p0_ref, p1_ref, b1a_ref, w1b_ref, b1b_ref, w2a_ref,
              t_ref):
    z = y_ref[...] + p0_ref[...] + p1_ref[...] + b1a_ref[...]
    z = jnp.maximum(z, 0.0)
    h = jnp.dot(z, w1b_ref[...], preferred_element_type=jnp.float32)
    h = jnp.maximum(h + b1b_ref[...], 0.0)
    t_ref[...] = jnp.sum(h * w2a_ref[...], axis=1, keepdims=True)


def _tc_c(y, p0, p1, b1a, w1b, b1b, w2a_row):
    return pl.pallas_call(
        _mlp_body,
        grid=(N // RB,),
        in_specs=[
            pl.BlockSpec((RB, H), lambda i: (i, 0)),
            pl.BlockSpec((RB, H), lambda i: (i, 0)),
            pl.BlockSpec((RB, H), lambda i: (i, 0)),
            pl.BlockSpec((1, H), lambda i: (0, 0)),
            pl.BlockSpec((H, H), lambda i: (0, 0)),
            pl.BlockSpec((1, H), lambda i: (0, 0)),
            pl.BlockSpec((1, H), lambda i: (0, 0)),
        ],
        out_specs=pl.BlockSpec((RB, 1), lambda i: (i, 0)),
        out_shape=jax.ShapeDtypeStruct((N, 1), jnp.float32),
    )(y, p0, p1, b1a, w1b, b1b, w2a_row)


def _sc_scal_body(src_hbm, dst_hbm, t_hbm, z_hbm, out_hbm,
                  t_v, srcf, dst2d, valsf, acc_sh, sem):
    del sem
    c = lax.axis_index("c")
    s = lax.axis_index("s")
    wid = c * NS + s
    pltpu.sync_copy(z_hbm, acc_sh.at[pl.ds(s * 640, 640)])
    pltpu.sync_copy(t_hbm, t_v)
    pltpu.sync_copy(src_hbm.at[pl.ds(wid * EPW, EPW)], srcf)
    pltpu.sync_copy(dst_hbm.at[pl.ds(wid * NBLK, NBLK)], dst2d)
    plsc.subcore_barrier()

    def gather_step(m, _):
        idx = srcf[pl.ds(m * LANES, LANES)]
        valsf[pl.ds(m * LANES, LANES)] = plsc.load_gather(t_v, [idx])
        return _

    lax.fori_loop(0, EPW // LANES, gather_step, None)
    for j in range(NBLK):
        pltpu.sync_copy(valsf.at[pl.ds(j * 128, 128)],
                        acc_sh.at[dst2d.at[j]], add=True)
    plsc.subcore_barrier()
    pltpu.sync_copy(acc_sh.at[pl.ds(s * 640, 640)],
                    out_hbm.at[c, 0, pl.ds(s * 640, 640)])


def _sc_d(src_flat, dst2d, t_pad, zd):
    mesh = plsc.VectorSubcoreMesh(core_axis_name="c", subcore_axis_name="s",
                                  num_cores=NC, num_subcores=NS)
    f = functools.partial(
        pl.kernel, mesh=mesh,
        compiler_params=pltpu.CompilerParams(use_tc_tiling_on_sc=False,
                                             needs_layout_passes=False),
        out_type=jax.ShapeDtypeStruct((NC, 1, NPAD), jnp.float32),
        scratch_types=[
            pltpu.VMEM((NPAD,), jnp.float32),
            pltpu.VMEM((EPW,), jnp.int32),
            pltpu.VMEM((NBLK, 128), jnp.int32),
            pltpu.VMEM((EPW,), jnp.float32),
            pltpu.VMEM_SHARED((NPAD,), jnp.float32),
            pltpu.SemaphoreType.DMA,
        ],
    )(_sc_scal_body)
    return f(src_flat, dst2d, t_pad, zd)


def _pool_body(t_ref, q0_ref, q1_ref, b_ref, scal_ref, o_ref, pacc, cacc):
    i = pl.program_id(0)
    u = t_ref[0] + q0_ref[0] + q1_ref[0] + scal_ref[0]
    u = jnp.maximum(u, 0.0)
    v = u * scal_ref[1] + scal_ref[2]
    gid = lax.broadcasted_iota(jnp.int32, (G, RB), 0)
    m = (gid == b_ref[0]).astype(jnp.float32)
    psum = jnp.sum(m * v, axis=1, keepdims=True)
    csum = jnp.sum(m, axis=1, keepdims=True)

    @pl.when(i == 0)
    def _():
        pacc[...] = psum
        cacc[...] = csum

    @pl.when(i > 0)
    def _():
        pacc[...] += psum
        cacc[...] += csum

    @pl.when(i == (N // RB) - 1)
    def _():
        o_ref[...] = pacc[...] / jnp.maximum(cacc[...], 1.0)


def _tc_e(t3, q03, q13, b3, scal):
    return pl.pallas_call(
        _pool_body,
        grid=(N // RB,),
        in_specs=[
            pl.BlockSpec((1, 1, RB), lambda i: (i, 0, 0)),
            pl.BlockSpec((1, 1, RB), lambda i: (i, 0, 0)),
            pl.BlockSpec((1, 1, RB), lambda i: (i, 0, 0)),
            pl.BlockSpec((1, 1, RB), lambda i: (i, 0, 0)),
            pl.BlockSpec(memory_space=pltpu.SMEM),
        ],
        out_specs=pl.BlockSpec((G, 1), lambda i: (0, 0)),
        out_shape=jax.ShapeDtypeStruct((G, 1), jnp.float32),
        scratch_shapes=[
            pltpu.VMEM((G, 1), jnp.float32),
            pltpu.VMEM((G, 1), jnp.float32),
        ],
    )(t3, q03, q13, b3, scal)


def kernel(x, edge_index, batch, W1a, b1a, W1b, b1b, W2a, b2a, W2b, b2b):
    src = edge_index[0]
    dst = edge_index[1]
    npad = EP - E
    pad_i = jnp.arange(npad, dtype=jnp.int32)
    src_flat = jnp.concatenate([src, pad_i % 128])
    dst2d = jnp.concatenate([dst, N + (pad_i % 128)]).reshape(EP // 128, 128)
    zb = jnp.zeros((640, H), jnp.float32)
    zd = jnp.zeros((640,), jnp.float32)

    y = _tc_a(x, W1a)
    p = _sc_b(src_flat, dst2d, y, zb)[:, :N]
    t = _tc_c(y, p[0], p[1], b1a.reshape(1, H), W1b, b1b.reshape(1, H),
              W2a.reshape(1, H))
    t_flat = t[:, 0]
    t_pad = jnp.concatenate([t_flat, jnp.zeros((NPAD - N,), jnp.float32)])
    q = _sc_d(src_flat, dst2d, t_pad, zd)[:, 0, :]
    scal = jnp.stack([b2a[0], W2b[0, 0], b2b[0], jnp.float32(0)])
    nb = N // RB
    pooled = _tc_e(t_flat.reshape(nb, 1, RB),
                   q[0, :N].reshape(nb, 1, RB),
                   q[1, :N].reshape(nb, 1, RB),
                   batch.reshape(nb, 1, RB),
                   scal)
    return pooled


# speedup vs baseline: 17.7448x; 1.0479x over previous
import functools

import jax
import jax.numpy as jnp
from jax import lax
from jax.experimental import pallas as pl
from jax.experimental.pallas import tpu as pltpu
from jax.experimental.pallas import tpu_sc as plsc

NC = 2
NS = 16
NW = NC * NS
LANES = 16

N = 10000
E = 320000
D = 128
H = 64
G = 256

ROWS = E // 128
NRB = ROWS // NW
NPAD = 10240
RB = 1000


def _mm_body(x_ref, w_ref, o_ref):
    o_ref[...] = jnp.dot(x_ref[...], w_ref[...],
                         preferred_element_type=jnp.float32,
                         precision=lax.Precision.HIGHEST)


def _tc_a(x, w1a):
    return pl.pallas_call(
        _mm_body,
        grid=(N // RB,),
        in_specs=[
            pl.BlockSpec((RB, D), lambda i: (i, 0)),
            pl.BlockSpec((D, H), lambda i: (0, 0)),
        ],
        out_specs=pl.BlockSpec((RB, H), lambda i: (i, 0)),
        out_shape=jax.ShapeDtypeStruct((N, H), jnp.float32),
    )(x, w1a)


def _sc_rows_body(e2_hbm, y_hbm, z_hbm, out_hbm,
                  src_v, dst_v, rows0, rows1, acc_sh, sem0, sem1):
    c = lax.axis_index("c")
    s = lax.axis_index("s")
    w = c * NS + s
    base = w * NRB + jnp.minimum(w, 4)
    pltpu.sync_copy(z_hbm, acc_sh.at[pl.ds(s * 625, 625)])
    pltpu.sync_copy(e2_hbm.at[0, pl.ds(base, NRB)], src_v.at[pl.ds(0, NRB)])
    pltpu.sync_copy(e2_hbm.at[1, pl.ds(base, NRB)], dst_v.at[pl.ds(0, NRB)])

    @pl.when(w < 4)
    def _():
        pltpu.sync_copy(e2_hbm.at[0, pl.ds(base + NRB, 1)],
                        src_v.at[pl.ds(NRB, 1)])
        pltpu.sync_copy(e2_hbm.at[1, pl.ds(base + NRB, 1)],
                        dst_v.at[pl.ds(NRB, 1)])

    plsc.subcore_barrier()
    rows = (rows0, rows1)
    sems = (sem0, sem1)
    cps = [None, None]
    cps[0] = pltpu.async_copy(y_hbm.at[src_v.at[0]], rows0, sem0)
    for j in range(NRB):
        b = (j + 1) % 2
        if j + 1 < NRB:
            cps[b] = pltpu.async_copy(y_hbm.at[src_v.at[j + 1]],
                                      rows[b], sems[b])
        cps[j % 2].wait()
        pltpu.sync_copy(rows[j % 2], acc_sh.at[dst_v.at[j]], add=True)

    @pl.when(w < 4)
    def _():
        pltpu.async_copy(y_hbm.at[src_v.at[NRB]], rows0, sem0).wait()
        pltpu.sync_copy(rows0, acc_sh.at[dst_v.at[NRB]], add=True)

    plsc.subcore_barrier()
    pltpu.sync_copy(acc_sh.at[pl.ds(s * 625, 625)],
                    out_hbm.at[c, pl.ds(s * 625, 625)])


def _sc_b(e2, y, zb):
    mesh = plsc.VectorSubcoreMesh(core_axis_name="c", subcore_axis_name="s",
                                  num_cores=NC, num_subcores=NS)
    f = functools.partial(
        pl.kernel, mesh=mesh,
        compiler_params=pltpu.CompilerParams(use_tc_tiling_on_sc=False),
        out_type=jax.ShapeDtypeStruct((NC, N, H), jnp.float32),
        scratch_types=[
            pltpu.VMEM((NRB + 1, 128), jnp.int32),
            pltpu.VMEM((NRB + 1, 128), jnp.int32),
            pltpu.VMEM((128, H), jnp.float32),
            pltpu.VMEM((128, H), jnp.float32),
            pltpu.VMEM_SHARED((N, H), jnp.float32),
            pltpu.SemaphoreType.DMA,
            pltpu.SemaphoreType.DMA,
        ],
    )(_sc_rows_body)
    return f(e2, y, zb)


def _mlp_body(y_ref, p0_ref, p1_ref, b1a_ref, w1b_ref, b1b_ref, w2a_ref,
              t_ref):
    z = y_ref[...] + p0_ref[0] + p1_ref[0] + b1a_ref[...]
    z = jnp.maximum(z, 0.0)
    h = jnp.dot(z, w1b_ref[...], preferred_element_type=jnp.float32,
                precision=lax.Precision.HIGHEST)
    h = jnp.maximum(h + b1b_ref[...], 0.0)
    t_ref[...] = jnp.sum(h * w2a_ref[...], axis=1, keepdims=True)


def _tc_c(y, p, b1a, w1b, b1b, w2a_row):
    return pl.pallas_call(
        _mlp_body,
        grid=(N // RB,),
        in_specs=[
            pl.BlockSpec((RB, H), lambda i: (i, 0)),
            pl.BlockSpec((1, RB, H), lambda i: (0, i, 0)),
            pl.BlockSpec((1, RB, H), lambda i: (1, i, 0)),
            pl.BlockSpec((1, H), lambda i: (0, 0)),
            pl.BlockSpec((H, H), lambda i: (0, 0)),
            pl.BlockSpec((1, H), lambda i: (0, 0)),
            pl.BlockSpec((1, H), lambda i: (0, 0)),
        ],
        out_specs=pl.BlockSpec((RB, 1), lambda i: (i, 0)),
        out_shape=jax.ShapeDtypeStruct((N, 1), jnp.float32),
    )(y, p, p, b1a, w1b, b1b, w2a_row)


def _sc_scal_body(e2_hbm, t_hbm, z_hbm, out_hbm,
                  t_v, srcf, dst_v, valsf, acc_sh, sem):
    del sem
    c = lax.axis_index("c")
    s = lax.axis_index("s")
    w = c * NS + s
    base = w * NRB + jnp.minimum(w, 4)
    nrows = NRB + jnp.where(w < 4, 1, 0)
    pltpu.sync_copy(z_hbm, acc_sh.at[pl.ds(s * 640, 640)])
    pltpu.sync_copy(t_hbm, t_v)
    pltpu.sync_copy(e2_hbm.at[0, pl.ds(base, NRB)],
                    srcf.at[pl.ds(0, NRB)])
    pltpu.sync_copy(e2_hbm.at[1, pl.ds(base, NRB)],
                    dst_v.at[pl.ds(0, NRB)])

    @pl.when(w < 4)
    def _():
        pltpu.sync_copy(e2_hbm.at[0, pl.ds(base + NRB, 1)],
                        srcf.at[pl.ds(NRB, 1)])
        pltpu.sync_copy(e2_hbm.at[1, pl.ds(base + NRB, 1)],
                        dst_v.at[pl.ds(NRB, 1)])

    plsc.subcore_barrier()

    def gather_step(j, carry):
        for k in range(8):
            idx = srcf[j, pl.ds(k * LANES, LANES)]
            vals = plsc.load_gather(t_v, [idx])
            valsf[j, pl.ds(k * LANES, LANES)] = vals
        return carry

    lax.fori_loop(0, nrows, gather_step, None)
    for j in range(NRB):
        pltpu.sync_copy(valsf.at[j], acc_sh.at[dst_v.at[j]], add=True)

    @pl.when(w < 4)
    def _():
        pltpu.sync_copy(valsf.at[NRB], acc_sh.at[dst_v.at[NRB]], add=True)

    plsc.subcore_barrier()
    pltpu.sync_copy(acc_sh.at[pl.ds(s * 640, 640)],
                    out_hbm.at[c, 0, pl.ds(s * 640, 640)])


def _sc_d(e2, t, zd):
    mesh = plsc.VectorSubcoreMesh(core_axis_name="c", subcore_axis_name="s",
                                  num_cores=NC, num_subcores=NS)
    f = functools.partial(
        pl.kernel, mesh=mesh,
        compiler_params=pltpu.CompilerParams(use_tc_tiling_on_sc=False,
                                             needs_layout_passes=False),
        out_type=jax.ShapeDtypeStruct((NC, 1, NPAD), jnp.float32),
        scratch_types=[
            pltpu.VMEM((N,), jnp.float32),
            pltpu.VMEM((NRB + 1, 128), jnp.int32),
            pltpu.VMEM((NRB + 1, 128), jnp.int32),
            pltpu.VMEM((NRB + 1, 128), jnp.float32),
            pltpu.VMEM_SHARED((NPAD,), jnp.float32),
            pltpu.SemaphoreType.DMA,
        ],
    )(_sc_scal_body)
    return f(e2, t, zd)


def _pool_body(t_ref, q0_ref, q1_ref, b_ref, scal_ref, o_ref, pacc, cacc):
    i = pl.program_id(0)
    u = t_ref[0] + q0_ref[0] + q1_ref[0] + scal_ref[0]
    u = jnp.maximum(u, 0.0)
    v = u * scal_ref[1] + scal_ref[2]
    gid = lax.broadcasted_iota(jnp.int32, (G, RB), 0)
    m = (gid == b_ref[0]).astype(jnp.float32)
    psum = jnp.sum(m * v, axis=1, keepdims=True)
    csum = jnp.sum(m, axis=1, keepdims=True)

    @pl.when(i == 0)
    def _():
        pacc[...] = psum
        cacc[...] = csum

    @pl.when(i > 0)
    def _():
        pacc[...] += psum
        cacc[...] += csum

    @pl.when(i == (N // RB) - 1)
    def _():
        o_ref[...] = pacc[...] / jnp.maximum(cacc[...], 1.0)


def _tc_e(t2, q0, q1, b2, scal):
    return pl.pallas_call(
        _pool_body,
        grid=(N // RB,),
        in_specs=[
            pl.BlockSpec((1, 1, RB), lambda i: (i, 0, 0)),
            pl.BlockSpec((1, 1, RB), lambda i: (i, 0, 0)),
            pl.BlockSpec((1, 1, RB), lambda i: (i, 0, 0)),
            pl.BlockSpec((1, 1, RB), lambda i: (i, 0, 0)),
            pl.BlockSpec(memory_space=pltpu.SMEM),
        ],
        out_specs=pl.BlockSpec((G, 1), lambda i: (0, 0)),
        out_shape=jax.ShapeDtypeStruct((G, 1), jnp.float32),
        scratch_shapes=[
            pltpu.VMEM((G, 1), jnp.float32),
            pltpu.VMEM((G, 1), jnp.float32),
        ],
    )(t2, q0, q1, b2, scal)


def kernel(x, edge_index, batch, W1a, b1a, W1b, b1b, W2a, b2a, W2b, b2b):
    zb = jnp.zeros((625, H), jnp.float32)
    zd = jnp.zeros((640,), jnp.float32)
    e2 = edge_index.reshape(2, ROWS, 128)

    y = _tc_a(x, W1a)
    p = _sc_b(e2, y, zb)
    t = _tc_c(y, p, b1a.reshape(1, H), W1b, b1b.reshape(1, H),
              W2a.reshape(1, H))[:, 0]
    q = _sc_d(e2, t, zd)
    scal = jnp.stack([b2a[0], W2b[0, 0], b2b[0], jnp.float32(0)])
    nb = N // RB
    pooled = _tc_e(t.reshape(nb, 1, RB),
                   q[0, 0, :N].reshape(nb, 1, RB),
                   q[1, 0, :N].reshape(nb, 1, RB),
                   batch.reshape(nb, 1, RB), scal)
    return pooled


# speedup vs baseline: 19.2996x; 1.0876x over previous
import functools

import jax
import jax.numpy as jnp
from jax import lax
from jax.experimental import pallas as pl
from jax.experimental.pallas import tpu as pltpu
from jax.experimental.pallas import tpu_sc as plsc

NC = 2
NS = 16
NW = NC * NS
LANES = 16

N = 10000
E = 320000
D = 128
H = 64
G = 256

EPW = E // NW
NB = EPW // 128
TAIL = EPW - NB * 128
NPAD = 10240
RB = 1000
NBUF = 8


def _mm_body(x_ref, w_ref, o_ref):
    o_ref[...] = jnp.dot(x_ref[...], w_ref[...],
                         preferred_element_type=jnp.float32,
                         precision=lax.Precision.HIGHEST)


def _tc_a(x, w1a):
    return pl.pallas_call(
        _mm_body,
        grid=(N // RB,),
        in_specs=[
            pl.BlockSpec((RB, D), lambda i: (i, 0)),
            pl.BlockSpec((D, H), lambda i: (0, 0)),
        ],
        out_specs=pl.BlockSpec((RB, H), lambda i: (i, 0)),
        out_shape=jax.ShapeDtypeStruct((N, H), jnp.float32),
    )(x, w1a)


def _sc_rows_body(edge_hbm, y_hbm, z_hbm, out_hbm,
                  srcf, dstf, *bufs_and_sems):
    rows = bufs_and_sems[:NBUF]
    acc_sh = bufs_and_sems[NBUF]
    gsem = bufs_and_sems[NBUF + 1:2 * NBUF + 1]
    ssem = bufs_and_sems[2 * NBUF + 1:3 * NBUF + 1]
    c = lax.axis_index("c")
    s = lax.axis_index("s")
    base = (c * NS + s) * EPW
    pltpu.sync_copy(z_hbm, acc_sh.at[pl.ds(s * 625, 625)])
    pltpu.sync_copy(edge_hbm.at[0, pl.ds(base, EPW)], srcf)
    pltpu.sync_copy(edge_hbm.at[1, pl.ds(base, EPW)], dstf)
    plsc.subcore_barrier()

    def fire_gather(j):
        return pltpu.async_copy(
            y_hbm.at[srcf.at[pl.ds(j * 128, 128)]],
            rows[j % NBUF], gsem[j % NBUF])

    def fire_scatter(j):
        return pltpu.async_copy(
            rows[j % NBUF],
            acc_sh.at[dstf.at[pl.ds(j * 128, 128)]],
            ssem[j % NBUF], add=True)

    LEAD = NBUF - 2
    gcp = [None] * NB
    scp = [None] * NB
    for j in range(LEAD):
        gcp[j] = fire_gather(j)
    for j in range(NB):
        k = j + LEAD
        if k < NB:
            if k >= NBUF:
                scp[k - NBUF].wait()
            gcp[k] = fire_gather(k)
        gcp[j].wait()
        scp[j] = fire_scatter(j)
    for j in range(NB - NBUF, NB):
        scp[j].wait()
    if TAIL:
        pltpu.async_copy(y_hbm.at[srcf.at[pl.ds(NB * 128, TAIL)]],
                         rows[0].at[pl.ds(0, TAIL)], gsem[0]).wait()
        pltpu.sync_copy(rows[0].at[pl.ds(0, TAIL)],
                        acc_sh.at[dstf.at[pl.ds(NB * 128, TAIL)]], add=True)
    plsc.subcore_barrier()
    pltpu.sync_copy(acc_sh.at[pl.ds(s * 625, 625)],
                    out_hbm.at[c, pl.ds(s * 625, 625)])


def _sc_b(edge_index, y, zb):
    mesh = plsc.VectorSubcoreMesh(core_axis_name="c", subcore_axis_name="s",
                                  num_cores=NC, num_subcores=NS)
    f = functools.partial(
        pl.kernel, mesh=mesh,
        compiler_params=pltpu.CompilerParams(use_tc_tiling_on_sc=False),
        out_type=jax.ShapeDtypeStruct((NC, N, H), jnp.float32),
        scratch_types=(
            [pltpu.VMEM((EPW,), jnp.int32),
             pltpu.VMEM((EPW,), jnp.int32)]
            + [pltpu.VMEM((128, H), jnp.float32)] * NBUF
            + [pltpu.VMEM_SHARED((N, H), jnp.float32)]
            + [pltpu.SemaphoreType.DMA] * (2 * NBUF)
        ),
    )(_sc_rows_body)
    return f(edge_index, y, zb)


def _mlp_body(y_ref, p0_ref, p1_ref, b1a_ref, w1b_ref, b1b_ref, w2a_ref,
              t_ref):
    z = y_ref[...] + p0_ref[0] + p1_ref[0] + b1a_ref[...]
    z = jnp.maximum(z, 0.0)
    h = jnp.dot(z, w1b_ref[...], preferred_element_type=jnp.float32,
                precision=lax.Precision.HIGHEST)
    h = jnp.maximum(h + b1b_ref[...], 0.0)
    t_ref[...] = jnp.sum(h * w2a_ref[...], axis=1, keepdims=True)


def _tc_c(y, p, b1a, w1b, b1b, w2a_row):
    return pl.pallas_call(
        _mlp_body,
        grid=(N // RB,),
        in_specs=[
            pl.BlockSpec((RB, H), lambda i: (i, 0)),
            pl.BlockSpec((1, RB, H), lambda i: (0, i, 0)),
            pl.BlockSpec((1, RB, H), lambda i: (1, i, 0)),
            pl.BlockSpec((1, H), lambda i: (0, 0)),
            pl.BlockSpec((H, H), lambda i: (0, 0)),
            pl.BlockSpec((1, H), lambda i: (0, 0)),
            pl.BlockSpec((1, H), lambda i: (0, 0)),
        ],
        out_specs=pl.BlockSpec((RB, 1), lambda i: (i, 0)),
        out_shape=jax.ShapeDtypeStruct((N, 1), jnp.float32),
    )(y, p, p, b1a, w1b, b1b, w2a_row)


def _sc_scal_body(edge_hbm, t_hbm, z_hbm, out_hbm,
                  t_v, srcf, dstf, valsf, acc_sh, sem):
    c = lax.axis_index("c")
    s = lax.axis_index("s")
    base = (c * NS + s) * EPW
    pltpu.sync_copy(z_hbm, acc_sh.at[pl.ds(s * 640, 640)])
    pltpu.sync_copy(t_hbm, t_v)
    pltpu.sync_copy(edge_hbm.at[0, pl.ds(base, EPW)], srcf)
    pltpu.sync_copy(edge_hbm.at[1, pl.ds(base, EPW)], dstf)
    plsc.subcore_barrier()

    zero16 = jnp.zeros((LANES,), jnp.int32)

    def gather_step(m, carry):
        idx = srcf[pl.ds(m * LANES, LANES)]
        vals = plsc.load_gather(t_v, [idx, zero16])
        valsf[pl.ds(m * LANES, LANES)] = vals
        return carry

    lax.fori_loop(0, EPW // LANES, gather_step, None)
    cps = []
    for j in range(NB):
        cps.append(pltpu.async_copy(
            valsf.at[pl.ds(j * 128, 128)],
            acc_sh.at[dstf.at[pl.ds(j * 128, 128)]], sem, add=True))
    for cp in cps:
        cp.wait()
    if TAIL:
        pltpu.sync_copy(valsf.at[pl.ds(NB * 128, TAIL)],
                        acc_sh.at[dstf.at[pl.ds(NB * 128, TAIL)]], add=True)
    plsc.subcore_barrier()
    pltpu.sync_copy(acc_sh.at[pl.ds(s * 640, 640)],
                    out_hbm.at[c, 0, pl.ds(s * 640, 640)])


def _sc_d(edge_index, t, zd):
    mesh = plsc.VectorSubcoreMesh(core_axis_name="c", subcore_axis_name="s",
                                  num_cores=NC, num_subcores=NS)
    f = functools.partial(
        pl.kernel, mesh=mesh,
        compiler_params=pltpu.CompilerParams(use_tc_tiling_on_sc=False,
                                             needs_layout_passes=False),
        out_type=jax.ShapeDtypeStruct((NC, 1, NPAD), jnp.float32),
        scratch_types=[
            pltpu.VMEM((N, 1), jnp.float32),
            pltpu.VMEM((EPW,), jnp.int32),
            pltpu.VMEM((EPW,), jnp.int32),
            pltpu.VMEM((EPW,), jnp.float32),
            pltpu.VMEM_SHARED((NPAD,), jnp.float32),
            pltpu.SemaphoreType.DMA,
        ],
    )(_sc_scal_body)
    return f(edge_index, t, zd)


def _pool_body(t_ref, q0_ref, q1_ref, b_ref, scal_ref, o_ref, pacc, cacc):
    i = pl.program_id(0)
    u = t_ref[0] + q0_ref[0] + q1_ref[0] + scal_ref[0]
    u = jnp.maximum(u, 0.0)
    v = u * scal_ref[1] + scal_ref[2]
    gid = lax.broadcasted_iota(jnp.int32, (G, RB), 0)
    m = (gid == b_ref[0]).astype(jnp.float32)
    psum = jnp.sum(m * v, axis=1, keepdims=True)
    csum = jnp.sum(m, axis=1, keepdims=True)

    @pl.when(i == 0)
    def _():
        pacc[...] = psum
        cacc[...] = csum

    @pl.when(i > 0)
    def _():
        pacc[...] += psum
        cacc[...] += csum

    @pl.when(i == (N // RB) - 1)
    def _():
        o_ref[...] = pacc[...] / jnp.maximum(cacc[...], 1.0)


def _tc_e(t2, q0, q1, b2, scal):
    return pl.pallas_call(
        _pool_body,
        grid=(N // RB,),
        in_specs=[
            pl.BlockSpec((1, 1, RB), lambda i: (i, 0, 0)),
            pl.BlockSpec((1, 1, RB), lambda i: (i, 0, 0)),
            pl.BlockSpec((1, 1, RB), lambda i: (i, 0, 0)),
            pl.BlockSpec((1, 1, RB), lambda i: (i, 0, 0)),
            pl.BlockSpec(memory_space=pltpu.SMEM),
        ],
        out_specs=pl.BlockSpec((G, 1), lambda i: (0, 0)),
        out_shape=jax.ShapeDtypeStruct((G, 1), jnp.float32),
        scratch_shapes=[
            pltpu.VMEM((G, 1), jnp.float32),
            pltpu.VMEM((G, 1), jnp.float32),
        ],
    )(t2, q0, q1, b2, scal)


def kernel(x, edge_index, batch, W1a, b1a, W1b, b1b, W2a, b2a, W2b, b2b):
    zb = jnp.zeros((625, H), jnp.float32)
    zd = jnp.zeros((640,), jnp.float32)

    y = _tc_a(x, W1a)
    p = _sc_b(edge_index, y, zb)
    t = _tc_c(y, p, b1a.reshape(1, H), W1b, b1b.reshape(1, H),
              W2a.reshape(1, H))
    q = _sc_d(edge_index, t, zd)
    scal = jnp.stack([b2a[0], W2b[0, 0], b2b[0], jnp.float32(0)])
    nb = N // RB
    pooled = _tc_e(t.reshape(nb, 1, RB),
                   q[0, 0, :N].reshape(nb, 1, RB),
                   q[1, 0, :N].reshape(nb, 1, RB),
                   batch.reshape(nb, 1, RB), scal)
    return pooled
